# Initial kernel scaffold; baseline (speedup 1.0000x reference)
import jax
import jax.numpy as jnp
from jax.experimental import pallas as pl


def kernel(x_c, x_delta, w_body, b_body, w_orelse, b_orelse):
    raise NotImplementedError("write your pallas kernel here")


# speedup vs baseline: 3.9235x; 3.9235x over previous
import jax
import jax.numpy as jnp
from jax.experimental import pallas as pl

_EPS = 1e-12
_BR = 1024


def _body(xc_ref, xd_ref, wb_ref, bb_ref, wo_ref, bo_ref,
          c_ref, d_ref, a_ref):
    xc = xc_ref[...]
    xd = xd_ref[...]
    wb = wb_ref[...]
    bb = bb_ref[...]
    wo = wo_ref[...]
    bo = bo_ref[...]

    t_c = xc[:, 0:1]
    t_d = xd[:, 0:1]
    lo = t_c - t_d
    hi = t_c + t_d
    frac = jnp.minimum(1.0, (0.0 - lo) / ((hi - lo) + _EPS))
    a1 = jnp.where(hi <= 0.0, 1.0, jnp.where(lo > 0.0, 0.0, frac))
    a2 = 1.0 - a1

    upper_l = jnp.minimum(hi, 0.0)
    cL = (lo + upper_l) * 0.5
    dL = (upper_l - lo) * 0.5
    lower_r = jnp.maximum(lo, 0.0)
    cR = (lower_r + hi) * 0.5
    dR = (hi - lower_r) * 0.5

    col = jax.lax.broadcasted_iota(jnp.int32, xc.shape, 1)
    is0 = col == 0
    xl_c = jnp.where(is0, cL, xc)
    xl_d = jnp.where(is0, dL, xd)
    xr_c = jnp.where(is0, cR, xc)
    xr_d = jnp.where(is0, dR, xd)

    c1 = xl_c * wb + bb
    d1 = xl_d * jnp.abs(wb)
    c2 = xr_c * wo + bo

    amax = jnp.maximum(a1, a2)
    ap1 = a1 / (amax + _EPS)
    ap2 = a2 / (amax + _EPS)
    c_out = (a1 * c1 + a2 * c2) / (a1 + a2 + _EPS)
    nc1 = ap1 * c1 + (1.0 - ap1) * c_out
    nc2 = ap2 * c2 + (1.0 - ap2) * c_out
    nd1 = ap1 * d1
    nd2 = ap2 * c2
    nl = jnp.minimum(nc1 - nd1, nc2 - nd2)
    nr = jnp.maximum(nc1 + nd1, nc2 + nd2)

    c_ref[...] = (nl + nr) * 0.5
    d_ref[...] = (nr - nl) * 0.5
    a_ref[...] = jnp.minimum(1.0, a1 + a2)


def kernel(x_c, x_delta, w_body, b_body, w_orelse, b_orelse):
    n, d = x_c.shape
    wb = w_body.reshape(1, d)
    bb = b_body.reshape(1, d)
    wo = w_orelse.reshape(1, d)
    bo = b_orelse.reshape(1, d)
    grid = (n // _BR,)
    row_spec = pl.BlockSpec((_BR, d), lambda i: (i, 0))
    vec_spec = pl.BlockSpec((1, d), lambda i: (0, 0))
    out = pl.pallas_call(
        _body,
        grid=grid,
        in_specs=[row_spec, row_spec, vec_spec, vec_spec, vec_spec, vec_spec],
        out_specs=[row_spec, row_spec, pl.BlockSpec((_BR, 1), lambda i: (i, 0))],
        out_shape=[
            jax.ShapeDtypeStruct((n, d), jnp.float32),
            jax.ShapeDtypeStruct((n, d), jnp.float32),
            jax.ShapeDtypeStruct((n, 1), jnp.float32),
        ],
    )(x_c, x_delta, wb, bb, wo, bo)
    return tuple(out)
